# Initial kernel scaffold; baseline (speedup 1.0000x reference)
import jax
import jax.numpy as jnp
from jax.experimental import pallas as pl


def kernel(src, seg, word_table, position_table, segment_table, reversed_position_table, gamma, beta):
    raise NotImplementedError("write your pallas kernel here")


# speedup vs baseline: 3.0036x; 3.0036x over previous
import functools

import jax
import jax.numpy as jnp
from jax import lax
from jax.experimental import pallas as pl
from jax.experimental.pallas import tpu as pltpu
from jax.experimental.pallas import tpu_sc as plsc

NC, NS, LANES = 2, 16, 16
NW = NC * NS


def _fast_rsqrt(v):
    i = lax.bitcast_convert_type(v, jnp.int32)
    i = jnp.int32(0x5F3759DF) - lax.shift_right_logical(i, 1)
    y = lax.bitcast_convert_type(i, jnp.float32)
    for _ in range(3):
        y = y * (1.5 - 0.5 * v * y * y)
    return y


def _make_sc_kernel(B, L, E, eps):
    rows_per_w = B // NW
    H = L // 2
    nch = E // LANES
    nseg = L // LANES
    mesh = plsc.VectorSubcoreMesh(core_axis_name="c", subcore_axis_name="s")

    @functools.partial(
        pl.kernel,
        out_type=jax.ShapeDtypeStruct((B * L, E), jnp.float32),
        mesh=mesh,
        compiler_params=pltpu.CompilerParams(needs_layout_passes=False),
        scratch_types=[
            pltpu.VMEM((L, E), jnp.float32),
            pltpu.VMEM((E,), jnp.float32),
            pltpu.VMEM((E,), jnp.float32),
            pltpu.VMEM((H,), jnp.int32),
            pltpu.VMEM((H,), jnp.int32),
            pltpu.VMEM((H,), jnp.int32),
            pltpu.VMEM((H,), jnp.int32),
            pltpu.VMEM((H,), jnp.int32),
            pltpu.VMEM((H,), jnp.int32),
            pltpu.VMEM((H, E), jnp.float32),
            pltpu.VMEM((H, E), jnp.float32),
            pltpu.SemaphoreType.DMA,
            pltpu.SemaphoreType.DMA,
        ],
    )
    def k(src_hbm, seg_hbm, wt_hbm, pt_hbm, comb_hbm, g_hbm, b_hbm, out_hbm,
          pos_v, gam_v, bet_v, src_a, src_b, seg_a, seg_b, cid_a, cid_b,
          wbuf, cbuf, sem0, sem1):
        wid = lax.axis_index("s") * NC + lax.axis_index("c")
        pltpu.sync_copy(pt_hbm.at[pl.ds(0, L)], pos_v)
        pltpu.sync_copy(g_hbm, gam_v)
        pltpu.sync_copy(b_hbm, bet_v)

        def row_body(j, carry):
            base = (wid * rows_per_w + j) * L
            pltpu.sync_copy(src_hbm.at[pl.ds(base, H)], src_a)
            pltpu.sync_copy(src_hbm.at[pl.ds(base + H, H)], src_b)
            pltpu.sync_copy(seg_hbm.at[pl.ds(base, H)], seg_a)
            pltpu.sync_copy(seg_hbm.at[pl.ds(base + H, H)], seg_b)

            segs, tots = [], []
            for kk in range(nseg):
                ref = seg_a if kk < nseg // 2 else seg_b
                off = (kk % (nseg // 2)) * LANES
                ch = ref[pl.ds(off, LANES)].astype(jnp.float32)
                segs.append((off, ch))
                tots.append(jnp.sum(ch))
            prefix = [jnp.float32(0)]
            for kk in range(1, nseg):
                prefix.append(prefix[-1] + tots[kk - 1])
            total = prefix[-1] + tots[-1]
            for kk in range(nseg):
                off, ch = segs[kk]
                rev = total - (plsc.cumsum(ch) + prefix[kk]) + ch
                dest = cid_a if kk < nseg // 2 else cid_b
                dest[pl.ds(off, LANES)] = (rev * 2 + ch).astype(jnp.int32)

            for h, (sref, cref) in enumerate(((src_a, cid_a), (src_b, cid_b))):
                cw = pltpu.async_copy(wt_hbm.at[sref], wbuf, sem0)
                cc = pltpu.async_copy(comb_hbm.at[cref], cbuf, sem1)
                cw.wait()
                cc.wait()

                def tok_body(t, c2, h=h):
                    pos = h * H + t
                    xs = []
                    for c in range(nch):
                        sl = pl.ds(c * LANES, LANES)
                        xs.append(wbuf[t, sl] + cbuf[t, sl] + pos_v[pos, sl])
                    sv = xs[0]
                    for c in range(1, nch):
                        sv = sv + xs[c]
                    qv = xs[0] * xs[0]
                    for c in range(1, nch):
                        qv = qv + xs[c] * xs[c]
                    mean = jnp.sum(sv) * (1.0 / E)
                    var = jnp.sum(qv) * (1.0 / E) - mean * mean
                    inv = _fast_rsqrt(var + eps)
                    for c in range(nch):
                        sl = pl.ds(c * LANES, LANES)
                        wbuf[t, sl] = (xs[c] - mean) * inv * gam_v[sl] + bet_v[sl]
                    return c2

                lax.fori_loop(0, H, tok_body, 0)
                pltpu.sync_copy(wbuf, out_hbm.at[pl.ds(base + h * H, H)])
            return carry

        lax.fori_loop(0, rows_per_w, row_body, 0)

    return k


def kernel(src, seg, word_table, position_table, segment_table,
           reversed_position_table, gamma, beta):
    B, L = src.shape
    E = word_table.shape[1]
    comb = (reversed_position_table[:, None, :]
            + segment_table[None, :2, :]).reshape(-1, E)
    k = _make_sc_kernel(B, L, E, 1e-6)
    out = k(src.reshape(-1), seg.reshape(-1), word_table, position_table,
            comb, gamma, beta)
    return out.reshape(B, L, E)


# speedup vs baseline: 6.4834x; 2.1586x over previous
import functools

import jax
import jax.numpy as jnp
from jax import lax
from jax.experimental import pallas as pl
from jax.experimental.pallas import tpu as pltpu
from jax.experimental.pallas import tpu_sc as plsc

NC, NS, LANES = 2, 16, 16
NW = NC * NS


def _fast_rsqrt(v):
    i = lax.bitcast_convert_type(v, jnp.int32)
    i = jnp.int32(0x5F3759DF) - lax.shift_right_logical(i, 1)
    y = lax.bitcast_convert_type(i, jnp.float32)
    for _ in range(3):
        y = y * (1.5 - 0.5 * v * y * y)
    return y


def _make_sc_kernel(B, L, E, eps):
    rows_per_w = B // NW
    H = L // 2
    nch = E // LANES
    nseg = L // LANES
    mesh = plsc.VectorSubcoreMesh(core_axis_name="c", subcore_axis_name="s")

    @functools.partial(
        pl.kernel,
        out_type=jax.ShapeDtypeStruct((B * L, E), jnp.float32),
        mesh=mesh,
        compiler_params=pltpu.CompilerParams(needs_layout_passes=False),
        scratch_types=[
            pltpu.VMEM((L, E), jnp.float32),
            pltpu.VMEM((E,), jnp.float32),
            pltpu.VMEM((E,), jnp.float32),
            pltpu.VMEM((H,), jnp.int32),
            pltpu.VMEM((H,), jnp.int32),
            pltpu.VMEM((H,), jnp.int32),
            pltpu.VMEM((H,), jnp.int32),
            pltpu.VMEM((H,), jnp.int32),
            pltpu.VMEM((H,), jnp.int32),
            pltpu.VMEM((H, E), jnp.float32),
            pltpu.VMEM((H, E), jnp.float32),
            pltpu.VMEM((H, E), jnp.float32),
            pltpu.VMEM((H, E), jnp.float32),
            pltpu.SemaphoreType.DMA,
            pltpu.SemaphoreType.DMA,
            pltpu.SemaphoreType.DMA,
            pltpu.SemaphoreType.DMA,
        ],
    )
    def k(src_hbm, seg_hbm, wt_hbm, pt_hbm, comb_hbm, g_hbm, b_hbm, out_hbm,
          pos_v, gam_v, bet_v, src_a, src_b, seg_a, seg_b, cid_a, cid_b,
          wbuf_a, wbuf_b, cbuf_a, cbuf_b, sem_a, sem_b, sem_oa, sem_ob):
        wid = lax.axis_index("s") * NC + lax.axis_index("c")
        pltpu.sync_copy(pt_hbm.at[pl.ds(0, L)], pos_v)
        pltpu.sync_copy(g_hbm, gam_v)
        pltpu.sync_copy(b_hbm, bet_v)

        def row_body(j, carry):
            base = (wid * rows_per_w + j) * L
            pltpu.sync_copy(src_hbm.at[pl.ds(base, H)], src_a)
            pltpu.sync_copy(src_hbm.at[pl.ds(base + H, H)], src_b)
            pltpu.sync_copy(seg_hbm.at[pl.ds(base, H)], seg_a)
            pltpu.sync_copy(seg_hbm.at[pl.ds(base + H, H)], seg_b)

            segs, tots = [], []
            for kk in range(nseg):
                ref = seg_a if kk < nseg // 2 else seg_b
                off = (kk % (nseg // 2)) * LANES
                ch = ref[pl.ds(off, LANES)].astype(jnp.float32)
                segs.append((off, ch))
                tots.append(jnp.sum(ch))
            prefix = [jnp.float32(0)]
            for kk in range(1, nseg):
                prefix.append(prefix[-1] + tots[kk - 1])
            total = prefix[-1] + tots[-1]
            for kk in range(nseg):
                off, ch = segs[kk]
                rev = total - (plsc.cumsum(ch) + prefix[kk]) + ch
                dest = cid_a if kk < nseg // 2 else cid_b
                dest[pl.ds(off, LANES)] = (rev * 2 + ch).astype(jnp.int32)

            def process(wbuf, cbuf, h):
                @plsc.parallel_loop(0, H, unroll=4)
                def tok_body(t):
                    pos = h * H + t
                    xs = []
                    for c in range(nch):
                        sl = pl.ds(c * LANES, LANES)
                        xs.append(wbuf[t, sl] + cbuf[t, sl] + pos_v[pos, sl])
                    sv = xs[0]
                    for c in range(1, nch):
                        sv = sv + xs[c]
                    qv = xs[0] * xs[0]
                    for c in range(1, nch):
                        qv = qv + xs[c] * xs[c]
                    mean = jnp.sum(sv) * (1.0 / E)
                    var = jnp.sum(qv) * (1.0 / E) - mean * mean
                    inv = _fast_rsqrt(var + eps)
                    for c in range(nch):
                        sl = pl.ds(c * LANES, LANES)
                        wbuf[t, sl] = (xs[c] - mean) * inv * gam_v[sl] + bet_v[sl]

            ga_w = pltpu.async_copy(wt_hbm.at[src_a], wbuf_a, sem_a)
            ga_c = pltpu.async_copy(comb_hbm.at[cid_a], cbuf_a, sem_a)
            gb_w = pltpu.async_copy(wt_hbm.at[src_b], wbuf_b, sem_b)
            gb_c = pltpu.async_copy(comb_hbm.at[cid_b], cbuf_b, sem_b)
            ga_w.wait()
            ga_c.wait()
            process(wbuf_a, cbuf_a, 0)
            oa = pltpu.async_copy(wbuf_a, out_hbm.at[pl.ds(base, H)], sem_oa)
            gb_w.wait()
            gb_c.wait()
            process(wbuf_b, cbuf_b, 1)
            ob = pltpu.async_copy(wbuf_b, out_hbm.at[pl.ds(base + H, H)], sem_ob)
            oa.wait()
            ob.wait()
            return carry

        lax.fori_loop(0, rows_per_w, row_body, 0)

    return k


def kernel(src, seg, word_table, position_table, segment_table,
           reversed_position_table, gamma, beta):
    B, L = src.shape
    E = word_table.shape[1]
    comb = (reversed_position_table[:, None, :]
            + segment_table[None, :2, :]).reshape(-1, E)
    k = _make_sc_kernel(B, L, E, 1e-6)
    out = k(src.reshape(-1), seg.reshape(-1), word_table, position_table,
            comb, gamma, beta)
    return out.reshape(B, L, E)


# speedup vs baseline: 6.8689x; 1.0595x over previous
import functools

import jax
import jax.numpy as jnp
from jax import lax
from jax.experimental import pallas as pl
from jax.experimental.pallas import tpu as pltpu
from jax.experimental.pallas import tpu_sc as plsc

NC, NS, LANES = 2, 16, 16
NW = NC * NS


def _fast_rsqrt(v):
    i = lax.bitcast_convert_type(v, jnp.int32)
    i = jnp.int32(0x5F3759DF) - lax.shift_right_logical(i, 1)
    y = lax.bitcast_convert_type(i, jnp.float32)
    for _ in range(3):
        y = y * (1.5 - 0.5 * v * y * y)
    return y


def _make_sc_kernel(B, L, E, eps):
    rows_per_w = B // NW
    H = L // 2
    nch = E // LANES
    nseg = L // LANES
    mesh = plsc.VectorSubcoreMesh(core_axis_name="c", subcore_axis_name="s")

    @functools.partial(
        pl.kernel,
        out_type=jax.ShapeDtypeStruct((B * L, E), jnp.float32),
        mesh=mesh,
        compiler_params=pltpu.CompilerParams(needs_layout_passes=False),
        scratch_types=[
            pltpu.VMEM((L, E), jnp.float32),
            pltpu.VMEM((E,), jnp.float32),
            pltpu.VMEM((E,), jnp.float32),
            pltpu.VMEM((H,), jnp.int32),
            pltpu.VMEM((H,), jnp.int32),
            pltpu.VMEM((H,), jnp.int32),
            pltpu.VMEM((H,), jnp.int32),
            pltpu.VMEM((H,), jnp.int32),
            pltpu.VMEM((H,), jnp.int32),
            pltpu.VMEM((H, E), jnp.float32),
            pltpu.VMEM((H, E), jnp.float32),
            pltpu.VMEM((H, E), jnp.float32),
            pltpu.VMEM((H, E), jnp.float32),
            pltpu.SemaphoreType.DMA,
            pltpu.SemaphoreType.DMA,
            pltpu.SemaphoreType.DMA,
            pltpu.SemaphoreType.DMA,
        ],
    )
    def k(src_hbm, seg_hbm, wt_hbm, pt_hbm, comb_hbm, g_hbm, b_hbm, out_hbm,
          pos_v, gam_v, bet_v, src_a, src_b, seg_a, seg_b, cid_a, cid_b,
          wbuf_a, wbuf_b, cbuf_a, cbuf_b, sem_a, sem_b, sem_oa, sem_ob):
        wid = lax.axis_index("s") * NC + lax.axis_index("c")
        pltpu.sync_copy(pt_hbm.at[pl.ds(0, L)], pos_v)
        pltpu.sync_copy(g_hbm, gam_v)
        pltpu.sync_copy(b_hbm, bet_v)

        def row_body(j, carry):
            base = (wid * rows_per_w + j) * L
            pltpu.sync_copy(src_hbm.at[pl.ds(base, H)], src_a)
            pltpu.sync_copy(src_hbm.at[pl.ds(base + H, H)], src_b)
            pltpu.sync_copy(seg_hbm.at[pl.ds(base, H)], seg_a)
            pltpu.sync_copy(seg_hbm.at[pl.ds(base + H, H)], seg_b)

            segs, tots = [], []
            for kk in range(nseg):
                ref = seg_a if kk < nseg // 2 else seg_b
                off = (kk % (nseg // 2)) * LANES
                ch = ref[pl.ds(off, LANES)].astype(jnp.float32)
                segs.append((off, ch))
                tots.append(jnp.sum(ch))
            prefix = [jnp.float32(0)]
            for kk in range(1, nseg):
                prefix.append(prefix[-1] + tots[kk - 1])
            total = prefix[-1] + tots[-1]
            for kk in range(nseg):
                off, ch = segs[kk]
                rev = total - (plsc.cumsum(ch) + prefix[kk]) + ch
                dest = cid_a if kk < nseg // 2 else cid_b
                dest[pl.ds(off, LANES)] = (rev * 2 + ch).astype(jnp.int32)

            def process(wbuf, cbuf, h):
                gams = [gam_v[pl.ds(c * LANES, LANES)] for c in range(nch)]
                bets = [bet_v[pl.ds(c * LANES, LANES)] for c in range(nch)]

                @plsc.parallel_loop(0, H, unroll=8)
                def tok_body(t):
                    pos = h * H + t
                    xs = []
                    for c in range(nch):
                        sl = pl.ds(c * LANES, LANES)
                        xs.append(wbuf[t, sl] + cbuf[t, sl] + pos_v[pos, sl])
                    sv = xs[0]
                    for c in range(1, nch):
                        sv = sv + xs[c]
                    qv = xs[0] * xs[0]
                    for c in range(1, nch):
                        qv = qv + xs[c] * xs[c]
                    mean = jnp.sum(sv) * (1.0 / E)
                    var = jnp.sum(qv) * (1.0 / E) - mean * mean
                    inv = _fast_rsqrt(var + eps)
                    for c in range(nch):
                        sl = pl.ds(c * LANES, LANES)
                        wbuf[t, sl] = (xs[c] - mean) * inv * gams[c] + bets[c]

            ga_w = pltpu.async_copy(wt_hbm.at[src_a], wbuf_a, sem_a)
            ga_c = pltpu.async_copy(comb_hbm.at[cid_a], cbuf_a, sem_a)
            gb_w = pltpu.async_copy(wt_hbm.at[src_b], wbuf_b, sem_b)
            gb_c = pltpu.async_copy(comb_hbm.at[cid_b], cbuf_b, sem_b)
            ga_w.wait()
            ga_c.wait()
            process(wbuf_a, cbuf_a, 0)
            oa = pltpu.async_copy(wbuf_a, out_hbm.at[pl.ds(base, H)], sem_oa)
            gb_w.wait()
            gb_c.wait()
            process(wbuf_b, cbuf_b, 1)
            ob = pltpu.async_copy(wbuf_b, out_hbm.at[pl.ds(base + H, H)], sem_ob)
            oa.wait()
            ob.wait()
            return carry

        lax.fori_loop(0, rows_per_w, row_body, 0)

    return k


def kernel(src, seg, word_table, position_table, segment_table,
           reversed_position_table, gamma, beta):
    B, L = src.shape
    E = word_table.shape[1]
    comb = (reversed_position_table[:, None, :]
            + segment_table[None, :2, :]).reshape(-1, E)
    k = _make_sc_kernel(B, L, E, 1e-6)
    out = k(src.reshape(-1), seg.reshape(-1), word_table, position_table,
            comb, gamma, beta)
    return out.reshape(B, L, E)
